# Initial kernel scaffold; baseline (speedup 1.0000x reference)
import jax
import jax.numpy as jnp
from jax.experimental import pallas as pl


def kernel(x, Wr, br, Wn, bn, W1, b1, W2, b2):
    raise NotImplementedError("write your pallas kernel here")


# speedup vs baseline: 1.2749x; 1.2749x over previous
import jax
import jax.numpy as jnp
from jax import lax
from jax.experimental import pallas as pl

N = 4096
D = 1024
E = 8
K = 2
BN = 512


def _moe_body(x_ref, wn_ref, bnr_ref, w1_ref, b1_ref, w2_ref, b2_ref, out_ref):
    e = pl.program_id(1)
    x = x_ref[...]
    nl = lax.dot_general(x, wn_ref[...], (((1,), (1,)), ((), ())),
                         preferred_element_type=jnp.float32) + bnr_ref[...]
    col = lax.broadcasted_iota(jnp.int32, nl.shape, 1)
    v1 = jnp.max(nl, axis=1, keepdims=True)
    i1 = jnp.min(jnp.where(nl == v1, col, E), axis=1, keepdims=True)
    nl2 = jnp.where(col == i1, -jnp.inf, nl)
    v2 = jnp.max(nl2, axis=1, keepdims=True)
    i2 = jnp.min(jnp.where(nl2 == v2, col, E), axis=1, keepdims=True)
    e2 = jnp.exp(v2 - v1)
    denom = 1.0 + e2
    gate = jnp.where(i1 == e, 1.0 / denom, 0.0) + jnp.where(i2 == e, e2 / denom, 0.0)

    h = jnp.maximum(
        lax.dot_general(x, w1_ref[0], (((1,), (1,)), ((), ())),
                        preferred_element_type=jnp.float32) + b1_ref[0], 0.0)
    eo = jnp.sum(h * w2_ref[0], axis=1, keepdims=True) + b2_ref[0, 0, 0]
    contrib = eo * gate

    @pl.when(e == 0)
    def _():
        out_ref[...] = contrib

    @pl.when(e != 0)
    def _():
        out_ref[...] += contrib


def kernel(x, Wr, br, Wn, bn, W1, b1, W2, b2):
    del Wr, br
    bnr = bn.reshape(1, E)
    b1r = b1.reshape(E, 1, D)
    w2r = W2.reshape(E, 1, D)
    b2r = b2.reshape(E, 1, 1)
    out = pl.pallas_call(
        _moe_body,
        grid=(N // BN, E),
        in_specs=[
            pl.BlockSpec((BN, D), lambda i, e: (i, 0)),
            pl.BlockSpec((E, D), lambda i, e: (0, 0)),
            pl.BlockSpec((1, E), lambda i, e: (0, 0)),
            pl.BlockSpec((1, D, D), lambda i, e: (e, 0, 0)),
            pl.BlockSpec((1, 1, D), lambda i, e: (e, 0, 0)),
            pl.BlockSpec((1, 1, D), lambda i, e: (e, 0, 0)),
            pl.BlockSpec((1, 1, 1), lambda i, e: (e, 0, 0)),
        ],
        out_specs=pl.BlockSpec((BN, 1), lambda i, e: (i, 0)),
        out_shape=jax.ShapeDtypeStruct((N, 1), jnp.float32),
    )(x, Wn, bnr, W1, b1r, w2r, b2r)
    return out
